# baseline (speedup 1.0000x reference)
import functools

import jax
import jax.numpy as jnp
from jax import lax
from jax.experimental import pallas as pl
from jax.experimental.pallas import tpu as pltpu
from jax.experimental.pallas import tpu_sc as plsc

HIDDEN = 64
N_NODES = 100000
E = 1000000
D = 8

_NC = 2
_NS = 16
_NW = _NC * _NS
_CH = 1280
_NCHUNK = (E - 320) // _CH
_NPAIR = _NCHUNK // 2
_NFULL = 2 * _NPAIR
_DROUNDS = _NPAIR // _NW // 2 * 2 // 2
_XPAIR = _NPAIR - 2 * _DROUNDS * _NW
_TPC = _CH // 128
_RG = 320
_RGB = 384
_TROWS = E * D // (D * 128) + 1

_KP = 16
_XW = _KP * HIDDEN
_TC_BLK = 1256


def _proj_body(xu_ref, xm_ref, wku_ref, wkm_ref, b_ref, u_ref, m_ref):
    xu = xu_ref[...].reshape(_TC_BLK, _XW)
    xm = xm_ref[...].reshape(_TC_BLK, _XW)
    u_ref[...] = jnp.dot(
        xu, wku_ref[...], preferred_element_type=jnp.float32,
    ) + b_ref[...]
    m_ref[...] = jnp.dot(
        xm, wkm_ref[...], preferred_element_type=jnp.float32,
    )


def _project(xu2, xm2, wku, wkm, bk):
    grid = -(-(N_NODES // _KP) // _TC_BLK)
    return pl.pallas_call(
        _proj_body,
        grid=(grid,),
        in_specs=[
            pl.BlockSpec((8 * _TC_BLK, 128), lambda i: (i, 0)),
            pl.BlockSpec((8 * _TC_BLK, 128), lambda i: (i, 0)),
            pl.BlockSpec((_XW, 128), lambda i: (0, 0)),
            pl.BlockSpec((_XW, 128), lambda i: (0, 0)),
            pl.BlockSpec((1, 128), lambda i: (0, 0)),
        ],
        out_specs=[
            pl.BlockSpec((_TC_BLK, 128), lambda i: (i, 0)),
            pl.BlockSpec((_TC_BLK, 128), lambda i: (i, 0)),
        ],
        out_shape=[
            jax.ShapeDtypeStruct((N_NODES // _KP, 128), jnp.float32),
            jax.ShapeDtypeStruct((N_NODES // _KP, 128), jnp.float32),
        ],
    )(xu2, xm2, wku, wkm, bk)


def _gather_add(u_tab, m_tab, idx):
    mesh = plsc.VectorSubcoreMesh(core_axis_name="c", subcore_axis_name="s")

    @functools.partial(
        pl.kernel,
        mesh=mesh,
        compiler_params=pltpu.CompilerParams(
            use_tc_tiling_on_sc=False, needs_layout_passes=False),
        out_type=jax.ShapeDtypeStruct((_TROWS, D, 128), jnp.float32),
        scratch_types=[
            pltpu.VMEM((2, 2 * _CH), jnp.int32),
            pltpu.VMEM((2, 2 * _CH), jnp.int32),
            pltpu.VMEM((_CH, D), jnp.float32),
            pltpu.VMEM((_CH, D), jnp.float32),
            pltpu.VMEM((_CH, D), jnp.float32),
            pltpu.VMEM((_CH, D), jnp.float32),
            pltpu.VMEM((_TPC, D, 128), jnp.float32),
            pltpu.VMEM((_TPC, D, 128), jnp.float32),
            pltpu.VMEM((_RG,), jnp.int32),
            pltpu.VMEM((_RG,), jnp.int32),
            pltpu.VMEM((_RGB, D), jnp.float32),
            pltpu.VMEM((_RGB, D), jnp.float32),
            pltpu.SemaphoreType.DMA,
            pltpu.SemaphoreType.DMA,
            pltpu.SemaphoreType.DMA,
            pltpu.SemaphoreType.DMA,
            pltpu.SemaphoreType.DMA,
            pltpu.SemaphoreType.DMA,
            pltpu.SemaphoreType.DMA,
            pltpu.SemaphoreType.DMA,
        ],
    )
    def k(u_hbm, m_hbm, idx_hbm, out_hbm, i0p, i1p, ua, ma, ub, mb,
          fa, fb, i0r, i1r, ur, mr,
          su_a, sm_a, su_b, sm_b, si0, si1, so_a, so_b):
        wid = lax.axis_index("s") * _NC + lax.axis_index("c")
        lane = lax.iota(jnp.int32, 16)

        def fire_idx(p, slot):
            base = p * (2 * _CH)
            c0 = pltpu.async_copy(
                idx_hbm.at[0, pl.ds(base, 2 * _CH)], i0p.at[slot], si0)
            c1 = pltpu.async_copy(
                idx_hbm.at[1, pl.ds(base, 2 * _CH)], i1p.at[slot], si1)
            return c0, c1

        def fire_gathers(slot, half, u_rows, m_rows, su, sm):
            off = half * _CH
            cu = pltpu.async_copy(
                u_hbm.at[i0p.at[slot, pl.ds(off, _CH)]], u_rows, su)
            cm = pltpu.async_copy(
                m_hbm.at[i1p.at[slot, pl.ds(off, _CH)]], m_rows, sm)
            return cu, cm

        def add_tiles(niter, u_rows, m_rows, flat):
            @plsc.parallel_loop(0, niter, step=1, unroll=8)
            def _vec(j):
                tt = j >> 6
                c = (j >> 3) & 7
                lg = j & 7
                r = tt * 128 + lg * 16 + lane
                cv = jnp.full((16,), c, jnp.int32)
                sv = (plsc.load_gather(u_rows, [r, cv])
                      + plsc.load_gather(m_rows, [r, cv]))
                flat[tt, c, pl.ds(lg * 16, 16)] = sv

        def fire_store(c, flat, so):
            return pltpu.async_copy(
                flat, out_hbm.at[pl.ds(c * _TPC, _TPC)], so)

        def half_pair(p, slot, half, u_rows, m_rows, su, sm, flat, so,
                      wait_store):
            cu, cm = fire_gathers(slot, half, u_rows, m_rows, su, sm)
            if wait_store:
                pltpu.make_async_copy(
                    flat, out_hbm.at[pl.ds(0, _TPC)], so).wait()
            cu.wait()
            cm.wait()
            add_tiles(_CH * D // 16, u_rows, m_rows, flat)
            fire_store(2 * p + half, flat, so)

        def dround(kk, carry):
            p0 = wid + _NW * (2 * kk)
            p1 = p0 + _NW
            c0, c1 = fire_idx(p0, 0)
            c0.wait()
            c1.wait()
            cu0, cm0 = fire_gathers(0, 0, ua, ma, su_a, sm_a)
            cu1, cm1 = fire_gathers(0, 1, ub, mb, su_b, sm_b)
            d0, d1 = fire_idx(p1, 1)
            cu0.wait()
            cm0.wait()
            add_tiles(_CH * D // 16, ua, ma, fa)
            fire_store(2 * p0, fa, so_a)
            cu1.wait()
            cm1.wait()
            add_tiles(_CH * D // 16, ub, mb, fb)
            fire_store(2 * p0 + 1, fb, so_b)
            d0.wait()
            d1.wait()
            half_pair(p1, 1, 0, ua, ma, su_a, sm_a, fa, so_a, True)
            half_pair(p1, 1, 1, ub, mb, su_b, sm_b, fb, so_b, True)
            pltpu.make_async_copy(
                fa, out_hbm.at[pl.ds(0, _TPC)], so_a).wait()
            pltpu.make_async_copy(
                fb, out_hbm.at[pl.ds(0, _TPC)], so_b).wait()
            return carry

        lax.fori_loop(0, _DROUNDS, dround, 0)

        @pl.when(wid < _XPAIR)
        def _extra_pair():
            p = 2 * _DROUNDS * _NW + wid
            c0, c1 = fire_idx(p, 0)
            c0.wait()
            c1.wait()
            cu0, cm0 = fire_gathers(0, 0, ua, ma, su_a, sm_a)
            cu1, cm1 = fire_gathers(0, 1, ub, mb, su_b, sm_b)
            cu0.wait()
            cm0.wait()
            add_tiles(_CH * D // 16, ua, ma, fa)
            s0 = fire_store(2 * p, fa, so_a)
            cu1.wait()
            cm1.wait()
            add_tiles(_CH * D // 16, ub, mb, fb)
            s1 = fire_store(2 * p + 1, fb, so_b)
            s0.wait()
            s1.wait()

        @pl.when(wid == _NW - 2)
        def _leftover():
            base = _NFULL * _CH
            pltpu.sync_copy(idx_hbm.at[0, pl.ds(base, _CH)],
                            i0p.at[0, pl.ds(0, _CH)])
            pltpu.sync_copy(idx_hbm.at[1, pl.ds(base, _CH)],
                            i1p.at[0, pl.ds(0, _CH)])
            cu, cm = fire_gathers(0, 0, ua, ma, su_a, sm_a)
            cu.wait()
            cm.wait()
            add_tiles(_CH * D // 16, ua, ma, fa)
            fire_store(_NFULL, fa, so_a).wait()

        @pl.when(wid == _NW - 1)
        def _ragged():
            pltpu.sync_copy(idx_hbm.at[0, pl.ds(E - _RG, _RG)], i0r)
            pltpu.sync_copy(idx_hbm.at[1, pl.ds(E - _RG, _RG)], i1r)
            cu = pltpu.async_copy(u_hbm.at[i0r], ur.at[pl.ds(0, _RG)], su_a)
            cm = pltpu.async_copy(m_hbm.at[i1r], mr.at[pl.ds(0, _RG)], sm_a)
            cu.wait()
            cm.wait()
            add_tiles(3 * D * 8, ur, mr, fa)
            pltpu.async_copy(fa.at[pl.ds(0, 3)],
                             out_hbm.at[pl.ds(_TROWS - 3, 3)], so_a).wait()

    return k(u_tab, m_tab, idx)


_FB = 632


def _declass_body(g_ref, o_ref):
    t = jnp.transpose(g_ref[...], (1, 0, 2))
    o_ref[...] = t.reshape(D, _FB * 128)[:7, :]


def _declassify(g3, ncls):
    return pl.pallas_call(
        _declass_body,
        grid=(-(-_TROWS // _FB),),
        in_specs=[pl.BlockSpec((_FB, D, 128), lambda i: (i, 0, 0))],
        out_specs=pl.BlockSpec((ncls, _FB * 128), lambda i: (0, i)),
        out_shape=jax.ShapeDtypeStruct((ncls, E), jnp.float32),
    )(g3)


def kernel(x_user, x_movie, edge_label_index, W, b):
    ncls = W.shape[0]
    idx = edge_label_index.astype(jnp.int32)
    wtu = jnp.zeros((HIDDEN, D), jnp.float32).at[:, :ncls].set(W[:, :HIDDEN].T)
    wtm = jnp.zeros((HIDDEN, D), jnp.float32).at[:, :ncls].set(W[:, HIDDEN:].T)
    eye = jnp.eye(_KP, dtype=jnp.float32)
    wku = jnp.kron(eye, wtu)
    wkm = jnp.kron(eye, wtm)
    bp = jnp.zeros((D,), jnp.float32).at[:ncls].set(b)
    bk = jnp.tile(bp, _KP).reshape(1, _KP * D)
    xu2 = x_user.reshape(N_NODES * HIDDEN // 128, 128)
    xm2 = x_movie.reshape(N_NODES * HIDDEN // 128, 128)
    u6, m6 = _project(xu2, xm2, wku, wkm, bk)
    u_tab = u6.reshape(N_NODES, D)
    m_tab = m6.reshape(N_NODES, D)
    g3 = _gather_add(u_tab, m_tab, idx)
    out_t = _declassify(g3, ncls)
    return out_t.T


# speedup vs baseline: 1.0008x; 1.0008x over previous
import functools

import jax
import jax.numpy as jnp
from jax import lax
from jax.experimental import pallas as pl
from jax.experimental.pallas import tpu as pltpu
from jax.experimental.pallas import tpu_sc as plsc

HIDDEN = 64
N_NODES = 100000
E = 1000000
D = 8

_NC = 2
_NS = 16
_NW = _NC * _NS
_CH = 1280
_NCHUNK = (E - 320) // _CH
_NPAIR = _NCHUNK // 2
_NFULL = 2 * _NPAIR
_DROUNDS = _NPAIR // _NW // 2 * 2 // 2
_XPAIR = _NPAIR - 2 * _DROUNDS * _NW
_TPC = _CH // 128
_RG = 320
_RGB = 384
_TROWS = E * D // (D * 128) + 1

_KP = 16
_XW = _KP * HIDDEN
_TC_BLK = 1256


def _proj_body(xu_ref, xm_ref, wku_ref, wkm_ref, b_ref, u_ref, m_ref):
    xu = xu_ref[...].reshape(_TC_BLK, _XW)
    xm = xm_ref[...].reshape(_TC_BLK, _XW)
    u_ref[...] = jnp.dot(
        xu, wku_ref[...], preferred_element_type=jnp.float32,
    ) + b_ref[...]
    m_ref[...] = jnp.dot(
        xm, wkm_ref[...], preferred_element_type=jnp.float32,
    )


def _project(xu2, xm2, wku, wkm, bk):
    grid = -(-(N_NODES // _KP) // _TC_BLK)
    return pl.pallas_call(
        _proj_body,
        grid=(grid,),
        in_specs=[
            pl.BlockSpec((8 * _TC_BLK, 128), lambda i: (i, 0)),
            pl.BlockSpec((8 * _TC_BLK, 128), lambda i: (i, 0)),
            pl.BlockSpec((_XW, 128), lambda i: (0, 0)),
            pl.BlockSpec((_XW, 128), lambda i: (0, 0)),
            pl.BlockSpec((1, 128), lambda i: (0, 0)),
        ],
        out_specs=[
            pl.BlockSpec((_TC_BLK, 128), lambda i: (i, 0)),
            pl.BlockSpec((_TC_BLK, 128), lambda i: (i, 0)),
        ],
        out_shape=[
            jax.ShapeDtypeStruct((N_NODES // _KP, 128), jnp.float32),
            jax.ShapeDtypeStruct((N_NODES // _KP, 128), jnp.float32),
        ],
    )(xu2, xm2, wku, wkm, bk)


def _gather_add(u_tab, m_tab, idx):
    mesh = plsc.VectorSubcoreMesh(core_axis_name="c", subcore_axis_name="s")

    @functools.partial(
        pl.kernel,
        mesh=mesh,
        compiler_params=pltpu.CompilerParams(
            use_tc_tiling_on_sc=False, needs_layout_passes=False),
        out_type=jax.ShapeDtypeStruct((_TROWS, D, 128), jnp.float32),
        scratch_types=[
            pltpu.VMEM((4, _CH), jnp.int32),
            pltpu.VMEM((4, _CH), jnp.int32),
            pltpu.VMEM((_CH, D), jnp.float32),
            pltpu.VMEM((_CH, D), jnp.float32),
            pltpu.VMEM((_CH, D), jnp.float32),
            pltpu.VMEM((_CH, D), jnp.float32),
            pltpu.VMEM((_TPC, D, 128), jnp.float32),
            pltpu.VMEM((_TPC, D, 128), jnp.float32),
            pltpu.VMEM((_RG,), jnp.int32),
            pltpu.VMEM((_RG,), jnp.int32),
            pltpu.VMEM((_RGB, D), jnp.float32),
            pltpu.VMEM((_RGB, D), jnp.float32),
            pltpu.SemaphoreType.DMA,
            pltpu.SemaphoreType.DMA,
            pltpu.SemaphoreType.DMA,
            pltpu.SemaphoreType.DMA,
            pltpu.SemaphoreType.DMA,
            pltpu.SemaphoreType.DMA,
            pltpu.SemaphoreType.DMA,
            pltpu.SemaphoreType.DMA,
        ],
    )
    def k(u_hbm, m_hbm, idx_hbm, out_hbm, i0p, i1p, ua, ma, ub, mb,
          fa, fb, i0r, i1r, ur, mr,
          su_a, sm_a, su_b, sm_b, si0, si1, so_a, so_b):
        wid = lax.axis_index("s") * _NC + lax.axis_index("c")
        lane = lax.iota(jnp.int32, 16)

        def fire_idx(p, slot):
            base = p * (2 * _CH)
            c0 = pltpu.async_copy(
                idx_hbm.at[0, pl.ds(base, _CH)], i0p.at[2 * slot], si0)
            c1 = pltpu.async_copy(
                idx_hbm.at[1, pl.ds(base, _CH)], i1p.at[2 * slot], si1)
            c2 = pltpu.async_copy(
                idx_hbm.at[0, pl.ds(base + _CH, _CH)],
                i0p.at[2 * slot + 1], si0)
            c3 = pltpu.async_copy(
                idx_hbm.at[1, pl.ds(base + _CH, _CH)],
                i1p.at[2 * slot + 1], si1)
            return (c0, c1, c2, c3)

        def fire_gathers(slot, half, u_rows, m_rows, su, sm):
            row = 2 * slot + half
            cu = pltpu.async_copy(u_hbm.at[i0p.at[row]], u_rows, su)
            cm = pltpu.async_copy(m_hbm.at[i1p.at[row]], m_rows, sm)
            return cu, cm

        def add_tiles(niter, u_rows, m_rows, flat):
            @plsc.parallel_loop(0, niter, step=1, unroll=8)
            def _vec(j):
                tt = j >> 6
                c = (j >> 3) & 7
                lg = j & 7
                r = tt * 128 + lg * 16 + lane
                cv = jnp.full((16,), c, jnp.int32)
                sv = (plsc.load_gather(u_rows, [r, cv])
                      + plsc.load_gather(m_rows, [r, cv]))
                flat[tt, c, pl.ds(lg * 16, 16)] = sv

        def fire_store(c, flat, so):
            return pltpu.async_copy(
                flat, out_hbm.at[pl.ds(c * _TPC, _TPC)], so)

        def half_pair(p, slot, half, u_rows, m_rows, su, sm, flat, so,
                      wait_store):
            cu, cm = fire_gathers(slot, half, u_rows, m_rows, su, sm)
            if wait_store:
                pltpu.make_async_copy(
                    flat, out_hbm.at[pl.ds(0, _TPC)], so).wait()
            cu.wait()
            cm.wait()
            add_tiles(_CH * D // 16, u_rows, m_rows, flat)
            fire_store(2 * p + half, flat, so)

        def dround(kk, carry):
            p0 = wid + _NW * (2 * kk)
            p1 = p0 + _NW
            for c in fire_idx(p0, 0):
                c.wait()
            cu0, cm0 = fire_gathers(0, 0, ua, ma, su_a, sm_a)
            cu1, cm1 = fire_gathers(0, 1, ub, mb, su_b, sm_b)
            dd = fire_idx(p1, 1)
            cu0.wait()
            cm0.wait()
            add_tiles(_CH * D // 16, ua, ma, fa)
            fire_store(2 * p0, fa, so_a)
            cu1.wait()
            cm1.wait()
            add_tiles(_CH * D // 16, ub, mb, fb)
            fire_store(2 * p0 + 1, fb, so_b)
            for c in dd:
                c.wait()
            half_pair(p1, 1, 0, ua, ma, su_a, sm_a, fa, so_a, True)
            half_pair(p1, 1, 1, ub, mb, su_b, sm_b, fb, so_b, True)
            pltpu.make_async_copy(
                fa, out_hbm.at[pl.ds(0, _TPC)], so_a).wait()
            pltpu.make_async_copy(
                fb, out_hbm.at[pl.ds(0, _TPC)], so_b).wait()
            return carry

        lax.fori_loop(0, _DROUNDS, dround, 0)

        @pl.when(wid < _XPAIR)
        def _extra_pair():
            p = 2 * _DROUNDS * _NW + wid
            for c in fire_idx(p, 0):
                c.wait()
            cu0, cm0 = fire_gathers(0, 0, ua, ma, su_a, sm_a)
            cu1, cm1 = fire_gathers(0, 1, ub, mb, su_b, sm_b)
            cu0.wait()
            cm0.wait()
            add_tiles(_CH * D // 16, ua, ma, fa)
            s0 = fire_store(2 * p, fa, so_a)
            cu1.wait()
            cm1.wait()
            add_tiles(_CH * D // 16, ub, mb, fb)
            s1 = fire_store(2 * p + 1, fb, so_b)
            s0.wait()
            s1.wait()

        @pl.when(wid == _NW - 2)
        def _leftover():
            base = _NFULL * _CH
            pltpu.sync_copy(idx_hbm.at[0, pl.ds(base, _CH)], i0p.at[0])
            pltpu.sync_copy(idx_hbm.at[1, pl.ds(base, _CH)], i1p.at[0])
            cu, cm = fire_gathers(0, 0, ua, ma, su_a, sm_a)
            cu.wait()
            cm.wait()
            add_tiles(_CH * D // 16, ua, ma, fa)
            fire_store(_NFULL, fa, so_a).wait()

        @pl.when(wid == _NW - 1)
        def _ragged():
            pltpu.sync_copy(idx_hbm.at[0, pl.ds(E - _RG, _RG)], i0r)
            pltpu.sync_copy(idx_hbm.at[1, pl.ds(E - _RG, _RG)], i1r)
            cu = pltpu.async_copy(u_hbm.at[i0r], ur.at[pl.ds(0, _RG)], su_a)
            cm = pltpu.async_copy(m_hbm.at[i1r], mr.at[pl.ds(0, _RG)], sm_a)
            cu.wait()
            cm.wait()
            add_tiles(3 * D * 8, ur, mr, fa)
            pltpu.async_copy(fa.at[pl.ds(0, 3)],
                             out_hbm.at[pl.ds(_TROWS - 3, 3)], so_a).wait()

    return k(u_tab, m_tab, idx)


_FB = 632


def _declass_body(g_ref, o_ref):
    t = jnp.transpose(g_ref[...], (1, 0, 2))
    o_ref[...] = t.reshape(D, _FB * 128)[:7, :]


def _declassify(g3, ncls):
    return pl.pallas_call(
        _declass_body,
        grid=(-(-_TROWS // _FB),),
        in_specs=[pl.BlockSpec((_FB, D, 128), lambda i: (i, 0, 0))],
        out_specs=pl.BlockSpec((ncls, _FB * 128), lambda i: (0, i)),
        out_shape=jax.ShapeDtypeStruct((ncls, E), jnp.float32),
    )(g3)


def kernel(x_user, x_movie, edge_label_index, W, b):
    ncls = W.shape[0]
    idx = edge_label_index.astype(jnp.int32)
    wtu = jnp.zeros((HIDDEN, D), jnp.float32).at[:, :ncls].set(W[:, :HIDDEN].T)
    wtm = jnp.zeros((HIDDEN, D), jnp.float32).at[:, :ncls].set(W[:, HIDDEN:].T)
    eye = jnp.eye(_KP, dtype=jnp.float32)
    wku = jnp.kron(eye, wtu)
    wkm = jnp.kron(eye, wtm)
    bp = jnp.zeros((D,), jnp.float32).at[:ncls].set(b)
    bk = jnp.tile(bp, _KP).reshape(1, _KP * D)
    xu2 = x_user.reshape(N_NODES * HIDDEN // 128, 128)
    xm2 = x_movie.reshape(N_NODES * HIDDEN // 128, 128)
    u6, m6 = _project(xu2, xm2, wku, wkm, bk)
    u_tab = u6.reshape(N_NODES, D)
    m_tab = m6.reshape(N_NODES, D)
    g3 = _gather_add(u_tab, m_tab, idx)
    out_t = _declassify(g3, ncls)
    return out_t.T


# speedup vs baseline: 1.0568x; 1.0560x over previous
import functools

import jax
import jax.numpy as jnp
from jax import lax
from jax.experimental import pallas as pl
from jax.experimental.pallas import tpu as pltpu
from jax.experimental.pallas import tpu_sc as plsc

HIDDEN = 64
N_NODES = 100000
E = 1000000
D = 8

_NC = 2
_NS = 16
_NW = _NC * _NS
_CH = 1280
_NCHUNK = (E - 320) // _CH
_ROUNDS = _NCHUNK // _NW
_TAIL = _NCHUNK - _ROUNDS * _NW
_TPC = _CH // 128
_RG = 320
_RGB = 384
_TROWS = E * D // (D * 128) + 1

_KP = 16
_XW = _KP * HIDDEN
_TC_BLK = 1256


def _proj_body(xu_ref, xm_ref, wku_ref, wkm_ref, b_ref, u_ref, m_ref):
    xu = xu_ref[...].reshape(_TC_BLK, _XW)
    xm = xm_ref[...].reshape(_TC_BLK, _XW)
    u_ref[...] = jnp.dot(
        xu, wku_ref[...], preferred_element_type=jnp.float32,
    ) + b_ref[...]
    m_ref[...] = jnp.dot(
        xm, wkm_ref[...], preferred_element_type=jnp.float32,
    )


def _project(xu2, xm2, wku, wkm, bk):
    grid = -(-(N_NODES // _KP) // _TC_BLK)
    return pl.pallas_call(
        _proj_body,
        grid=(grid,),
        in_specs=[
            pl.BlockSpec((8 * _TC_BLK, 128), lambda i: (i, 0)),
            pl.BlockSpec((8 * _TC_BLK, 128), lambda i: (i, 0)),
            pl.BlockSpec((_XW, 128), lambda i: (0, 0)),
            pl.BlockSpec((_XW, 128), lambda i: (0, 0)),
            pl.BlockSpec((1, 128), lambda i: (0, 0)),
        ],
        out_specs=[
            pl.BlockSpec((_TC_BLK, 128), lambda i: (i, 0)),
            pl.BlockSpec((_TC_BLK, 128), lambda i: (i, 0)),
        ],
        out_shape=[
            jax.ShapeDtypeStruct((N_NODES // _KP, 128), jnp.float32),
            jax.ShapeDtypeStruct((N_NODES // _KP, 128), jnp.float32),
        ],
    )(xu2, xm2, wku, wkm, bk)


def _gather_add(u_tab, m_tab, idx):
    mesh = plsc.VectorSubcoreMesh(core_axis_name="c", subcore_axis_name="s")

    @functools.partial(
        pl.kernel,
        mesh=mesh,
        compiler_params=pltpu.CompilerParams(
            use_tc_tiling_on_sc=False, needs_layout_passes=False),
        out_type=jax.ShapeDtypeStruct((_TROWS, D, 128), jnp.float32),
        scratch_types=[
            pltpu.VMEM((2, _CH), jnp.int32),
            pltpu.VMEM((2, _CH), jnp.int32),
            pltpu.VMEM((_CH, D), jnp.float32),
            pltpu.VMEM((_CH, D), jnp.float32),
            pltpu.VMEM((_CH, D), jnp.float32),
            pltpu.VMEM((_CH, D), jnp.float32),
            pltpu.VMEM((_TPC, D, 128), jnp.float32),
            pltpu.VMEM((_TPC, D, 128), jnp.float32),
            pltpu.VMEM((_RG,), jnp.int32),
            pltpu.VMEM((_RG,), jnp.int32),
            pltpu.VMEM((_RGB, D), jnp.float32),
            pltpu.VMEM((_RGB, D), jnp.float32),
            pltpu.SemaphoreType.DMA,
            pltpu.SemaphoreType.DMA,
            pltpu.SemaphoreType.DMA,
            pltpu.SemaphoreType.DMA,
        ],
    )
    def k(u_hbm, m_hbm, idx_hbm, out_hbm, i0_v, i1_v, ua, ma, ub, mb,
          fa, fb, i0r, i1r, ur, mr, su_a, sm_a, su_b, sm_b):
        wid = lax.axis_index("s") * _NC + lax.axis_index("c")
        lane = lax.iota(jnp.int32, 16)

        def load_and_fire(c, slot, u_rows, m_rows, su, sm):
            base = c * _CH
            pltpu.sync_copy(idx_hbm.at[0, pl.ds(base, _CH)], i0_v.at[slot])
            pltpu.sync_copy(idx_hbm.at[1, pl.ds(base, _CH)], i1_v.at[slot])
            cu = pltpu.async_copy(u_hbm.at[i0_v.at[slot]], u_rows, su)
            cm = pltpu.async_copy(m_hbm.at[i1_v.at[slot]], m_rows, sm)
            return cu, cm

        def add_tiles(niter, u_rows, m_rows, flat):
            @plsc.parallel_loop(0, niter, step=1, unroll=8)
            def _vec(j):
                tt = j >> 6
                c = (j >> 3) & 7
                lg = j & 7
                r = tt * 128 + lg * 16 + lane
                cv = jnp.full((16,), c, jnp.int32)
                sv = (plsc.load_gather(u_rows, [r, cv])
                      + plsc.load_gather(m_rows, [r, cv]))
                flat[tt, c, pl.ds(lg * 16, 16)] = sv

        def add_and_store(c, u_rows, m_rows, flat):
            add_tiles(_CH * D // 16, u_rows, m_rows, flat)
            pltpu.sync_copy(flat, out_hbm.at[pl.ds(c * _TPC, _TPC)])

        def pair(j, carry):
            ca = wid + _NW * (2 * j)
            cb = wid + _NW * (2 * j + 1)
            cua, cma = load_and_fire(ca, 0, ua, ma, su_a, sm_a)
            cub, cmb = load_and_fire(cb, 1, ub, mb, su_b, sm_b)
            cua.wait()
            cma.wait()
            add_and_store(ca, ua, ma, fa)
            cub.wait()
            cmb.wait()
            add_and_store(cb, ub, mb, fb)
            return carry

        lax.fori_loop(0, _ROUNDS // 2, pair, 0)

        @pl.when(wid < _TAIL)
        def _fire_tail():
            load_and_fire(_ROUNDS * _NW + wid, 1, ub, mb, su_b, sm_b)

        @pl.when(wid == _NW - 1)
        def _fire_rag():
            pltpu.sync_copy(idx_hbm.at[0, pl.ds(E - _RG, _RG)], i0r)
            pltpu.sync_copy(idx_hbm.at[1, pl.ds(E - _RG, _RG)], i1r)
            pltpu.async_copy(u_hbm.at[i0r], ur.at[pl.ds(0, _RG)], su_a)
            pltpu.async_copy(m_hbm.at[i1r], mr.at[pl.ds(0, _RG)], sm_a)

        @pl.when(wid < _TAIL)
        def _do_tail():
            pltpu.make_async_copy(u_hbm.at[i0_v.at[1]], ub, su_b).wait()
            pltpu.make_async_copy(m_hbm.at[i1_v.at[1]], mb, sm_b).wait()
            add_and_store(_ROUNDS * _NW + wid, ub, mb, fb)

        @pl.when(wid == _NW - 1)
        def _do_rag():
            pltpu.make_async_copy(
                u_hbm.at[i0r], ur.at[pl.ds(0, _RG)], su_a).wait()
            pltpu.make_async_copy(
                m_hbm.at[i1r], mr.at[pl.ds(0, _RG)], sm_a).wait()
            add_tiles(3 * D * 8, ur, mr, fa)
            pltpu.sync_copy(fa.at[pl.ds(0, 3)],
                            out_hbm.at[pl.ds(_TROWS - 3, 3)])

    return k(u_tab, m_tab, idx)


_FB = 632


def _declass_body(g_ref, o_ref):
    t = jnp.transpose(g_ref[...], (1, 0, 2))
    o_ref[...] = t.reshape(D, _FB * 128)[:7, :]


def _declassify(g3, ncls):
    return pl.pallas_call(
        _declass_body,
        grid=(-(-_TROWS // _FB),),
        in_specs=[pl.BlockSpec((_FB, D, 128), lambda i: (i, 0, 0))],
        out_specs=pl.BlockSpec((ncls, _FB * 128), lambda i: (0, i)),
        out_shape=jax.ShapeDtypeStruct((ncls, E), jnp.float32),
    )(g3)


def kernel(x_user, x_movie, edge_label_index, W, b):
    ncls = W.shape[0]
    idx = edge_label_index.astype(jnp.int32)
    wtu = jnp.zeros((HIDDEN, D), jnp.float32).at[:, :ncls].set(W[:, :HIDDEN].T)
    wtm = jnp.zeros((HIDDEN, D), jnp.float32).at[:, :ncls].set(W[:, HIDDEN:].T)
    eye = jnp.eye(_KP, dtype=jnp.float32)
    wku = jnp.kron(eye, wtu)
    wkm = jnp.kron(eye, wtm)
    bp = jnp.zeros((D,), jnp.float32).at[:ncls].set(b)
    bk = jnp.tile(bp, _KP).reshape(1, _KP * D)
    xu2 = x_user.reshape(N_NODES * HIDDEN // 128, 128)
    xm2 = x_movie.reshape(N_NODES * HIDDEN // 128, 128)
    u6, m6 = _project(xu2, xm2, wku, wkm, bk)
    u_tab = u6.reshape(N_NODES, D)
    m_tab = m6.reshape(N_NODES, D)
    g3 = _gather_add(u_tab, m_tab, idx)
    out_t = _declassify(g3, ncls)
    return out_t.T
